# baseline (speedup 1.0000x reference)
import jax
import jax.numpy as jnp
from jax import lax
from jax.experimental import pallas as pl
from jax.experimental.pallas import tpu as pltpu, tpu_sc as plsc

_NC, _NS, _L = 2, 16, 16
_NW = _NC * _NS

_ROWS, _COLS, _D = 16384, 200, 4
_N = _ROWS * _COLS
_JT = _COLS // 8
_IT = _ROWS // 128
_ITB = _IT // _NW
_CHUNK = _ITB * 8 * 128
_OUTW = _CHUNK * _D
_JROW = _IT * _D * 128


def _lookup_body(idx_hbm, emb_hbm, out_hbm, emb_v, idx_v, out_v,
                 sem_in0, sem_in1, sem_out0, sem_out1):
    wid = lax.axis_index("s") * _NC + lax.axis_index("c")
    sem_in = (sem_in0, sem_in1)
    sem_out = (sem_out0, sem_out1)
    pltpu.sync_copy(emb_hbm, emb_v)

    def in_copy(k, p):
        return pltpu.make_async_copy(
            idx_hbm.at[pl.ds((k * _IT + _ITB * wid) * 1024, _CHUNK)],
            idx_v.at[p],
            sem_in[p],
        )

    lanes = lax.iota(jnp.int32, _L)
    cvec = [lanes + c * _L for c in range(_D)]

    def compute(k, p):
        for jn in range(8):
            @plsc.parallel_loop(0, 32, 1, unroll=8)
            def _(m):
                itl = m >> 3
                kk = m & 7
                iv = idx_v[p, pl.ds(itl * 1024 + jn * 128 + kk * 16, _L)]
                w64 = iv * (_D * _L)
                dst = jn * 2048 + itl * 512 + kk * 16
                for c in range(_D):
                    g = plsc.load_gather(emb_v, [w64 + cvec[c]])
                    out_v[p, pl.ds(dst + c * 128, _L)] = g

            pltpu.make_async_copy(
                out_v.at[p, pl.ds(jn * 2048, 2048)],
                out_hbm.at[pl.ds((8 * k + jn) * _JROW + 2048 * wid, 2048)],
                sem_out[p],
            ).start()

    def out_drain(p):
        pltpu.make_async_copy(
            out_hbm.at[pl.ds(0, _OUTW)], out_v.at[p], sem_out[p]
        ).wait()

    in_copy(0, 0).start()

    def pair(t, _):
        k0 = t * 2
        k1 = k0 + 1
        in_copy(k1, 1).start()
        in_copy(k0, 0).wait()

        @pl.when(t >= 1)
        def _():
            out_drain(0)

        compute(k0, 0)
        in_copy(k0 + 2, 0).start()
        in_copy(k1, 1).wait()

        @pl.when(t >= 1)
        def _():
            out_drain(1)

        compute(k1, 1)
        return 0

    lax.fori_loop(0, (_JT - 1) // 2, pair, 0)
    in_copy(_JT - 1, 0).wait()
    out_drain(0)
    compute(_JT - 1, 0)
    out_drain(1)
    out_drain(0)


def kernel(src_sent, emb):
    idx = src_sent.reshape(128, 128, _JT, 8).transpose(2, 0, 3, 1).reshape(-1)
    emb_flat = jnp.tile(emb.reshape(10, _D, 1), (1, 1, _L)).reshape(-1)
    mesh = plsc.VectorSubcoreMesh(core_axis_name="c", subcore_axis_name="s")
    out = pl.kernel(
        _lookup_body,
        out_type=jax.ShapeDtypeStruct((_N * _D,), jnp.float32),
        mesh=mesh,
        scratch_types=[
            pltpu.VMEM((10 * _D * _L,), jnp.float32),
            pltpu.VMEM((2, _CHUNK), jnp.int32),
            pltpu.VMEM((2, _OUTW), jnp.float32),
            pltpu.SemaphoreType.DMA,
            pltpu.SemaphoreType.DMA,
            pltpu.SemaphoreType.DMA,
            pltpu.SemaphoreType.DMA,
        ],
        compiler_params=pltpu.CompilerParams(needs_layout_passes=False),
    )(idx, emb_flat)
    return (
        out.reshape(_COLS, 128, _D, 128)
        .transpose(1, 3, 0, 2)
        .reshape(_ROWS, _COLS, _D)
    )


# speedup vs baseline: 1.4409x; 1.4409x over previous
import jax
import jax.numpy as jnp
from jax import lax
from jax.experimental import pallas as pl
from jax.experimental.pallas import tpu as pltpu, tpu_sc as plsc

_NC, _NS, _L = 2, 16, 16
_NW = _NC * _NS

_ROWS, _COLS, _D = 16384, 200, 4
_N = _ROWS * _COLS
_JT = _COLS // 8
_IT = _ROWS // 128
_ITB = _IT // _NW
_CHUNK = _ITB * 8 * 128
_OUTW = _CHUNK * _D
_JROW = _IT * _D * 128


def _lookup_body(idx_hbm, emb_hbm, out_hbm, emb_v, idx_v, out_v,
                 sem_in0, sem_in1, sem_out0, sem_out1):
    wid = lax.axis_index("s") * _NC + lax.axis_index("c")
    sem_in = (sem_in0, sem_in1)
    sem_out = (sem_out0, sem_out1)
    pltpu.sync_copy(emb_hbm, emb_v)

    def in_copy(k, p):
        return pltpu.make_async_copy(
            idx_hbm.at[pl.ds((k * _IT + _ITB * wid) * 1024, _CHUNK)],
            idx_v.at[p],
            sem_in[p],
        )

    lanes = lax.iota(jnp.int32, _L)
    cvec = [lanes + c * _L for c in range(_D)]

    def compute(k, p):
        @plsc.parallel_loop(0, _CHUNK // _L, 1, unroll=8)
        def _(m):
            itl = m >> 6
            jn = (m >> 3) & 7
            kk = m & 7
            iv = idx_v[p, pl.ds(itl * 1024 + jn * 128 + kk * 16, _L)]
            w64 = iv * (_D * _L)
            dst = jn * 2048 + itl * 512 + kk * 16
            for c in range(_D):
                g = plsc.load_gather(emb_v, [w64 + cvec[c]])
                out_v[p, pl.ds(dst + c * 128, _L)] = g

    def out_start(k, p):
        for jn in range(8):
            pltpu.make_async_copy(
                out_v.at[p, pl.ds(jn * 2048, 2048)],
                out_hbm.at[pl.ds((8 * k + jn) * _JROW + 2048 * wid, 2048)],
                sem_out[p],
            ).start()

    def out_drain(p):
        pltpu.make_async_copy(
            out_hbm.at[pl.ds(0, _OUTW)], out_v.at[p], sem_out[p]
        ).wait()

    in_copy(0, 0).start()

    def pair(t, _):
        k0 = t * 2
        k1 = k0 + 1
        in_copy(k1, 1).start()
        in_copy(k0, 0).wait()

        @pl.when(t >= 1)
        def _():
            out_drain(0)

        compute(k0, 0)
        out_start(k0, 0)
        in_copy(k0 + 2, 0).start()
        in_copy(k1, 1).wait()

        @pl.when(t >= 1)
        def _():
            out_drain(1)

        compute(k1, 1)
        out_start(k1, 1)
        return 0

    lax.fori_loop(0, (_JT - 1) // 2, pair, 0)
    in_copy(_JT - 1, 0).wait()
    out_drain(0)
    compute(_JT - 1, 0)
    out_start(_JT - 1, 0)
    out_drain(1)
    out_drain(0)


def kernel(src_sent, emb):
    idx = src_sent.reshape(128, 128, _JT, 8).transpose(2, 0, 3, 1).reshape(-1)
    emb_flat = jnp.tile(emb.reshape(10, _D, 1), (1, 1, _L)).reshape(-1)
    mesh = plsc.VectorSubcoreMesh(core_axis_name="c", subcore_axis_name="s")
    out = pl.kernel(
        _lookup_body,
        out_type=jax.ShapeDtypeStruct((_N * _D,), jnp.float32),
        mesh=mesh,
        scratch_types=[
            pltpu.VMEM((10 * _D * _L,), jnp.float32),
            pltpu.VMEM((2, _CHUNK), jnp.int32),
            pltpu.VMEM((2, _OUTW), jnp.float32),
            pltpu.SemaphoreType.DMA,
            pltpu.SemaphoreType.DMA,
            pltpu.SemaphoreType.DMA,
            pltpu.SemaphoreType.DMA,
        ],
        compiler_params=pltpu.CompilerParams(needs_layout_passes=False),
    )(idx, emb_flat)
    return (
        out.reshape(_COLS, 128, _D, 128)
        .transpose(1, 3, 0, 2)
        .reshape(_ROWS, _COLS, _D)
    )
